# baseline (speedup 1.0000x reference)
import functools

import jax
import jax.numpy as jnp
from jax.experimental import pallas as pl

Q = 512
K = 100000
D = 768
K_CHUNK = 100
K_DOC = 10
N_DOCS = 10000

KT = 2048
K_PAD = ((K + KT - 1) // KT) * KT
N_TILES = K_PAD // KT


def _proj_kernel(q_ref, w_ref, b_ref, out_ref):
    acc = jnp.dot(q_ref[...], w_ref[...], preferred_element_type=jnp.float32)
    out_ref[...] = jnp.tanh(acc + b_ref[...])


def _sim_kernel(q_ref, k_ref, out_ref):
    i = pl.program_id(0)
    sim = jax.lax.dot_general(
        q_ref[...], k_ref[...], (((1,), (1,)), ((), ())),
        preferred_element_type=jnp.float32)
    col = i * KT + jax.lax.broadcasted_iota(jnp.int32, (Q, KT), 1)
    out_ref[...] = jnp.where(col < K, sim, -3.4e38)


def _project(queries, W, b):
    return pl.pallas_call(
        _proj_kernel,
        out_shape=jax.ShapeDtypeStruct((Q, D), jnp.float32),
    )(queries, W, b.reshape(1, D))


def _sim(q, keys_pad):
    return pl.pallas_call(
        _sim_kernel,
        grid=(N_TILES,),
        in_specs=[
            pl.BlockSpec((Q, D), lambda i: (0, 0)),
            pl.BlockSpec((KT, D), lambda i: (i, 0)),
        ],
        out_specs=pl.BlockSpec((Q, KT), lambda i: (0, i)),
        out_shape=jax.ShapeDtypeStruct((Q, K_PAD), jnp.float32),
    )(q, keys_pad)


def kernel(queries, keys, doc_ids, W, b):
    q = _project(queries, W, b)
    keys_pad = jnp.pad(keys, ((0, K_PAD - K), (0, 0)))
    sim = _sim(q, keys_pad)
    _, inds = jax.lax.top_k(sim, K_CHUNK)
    chunks = jnp.take(keys, inds, axis=0)
    qn = q / (jnp.linalg.norm(q, axis=-1, keepdims=True) + 1e-8)
    cn = chunks / (jnp.linalg.norm(chunks, axis=-1, keepdims=True) + 1e-8)
    cos = jnp.sum(qn[:, None, :] * cn, axis=-1)
    scores = jnp.exp(cos)
    docs = jnp.take(doc_ids, inds, axis=0)
    row = jnp.arange(Q, dtype=jnp.int32)[:, None]
    doc_scores = jnp.zeros((Q, N_DOCS), dtype=jnp.float32)
    doc_scores = doc_scores.at[row, docs].add(scores)
    top_scores, top_docs = jax.lax.top_k(doc_scores, K_DOC)
    return top_scores, top_docs


# speedup vs baseline: 3.7220x; 3.7220x over previous
import functools

import jax
import jax.numpy as jnp
from jax import lax
from jax.experimental import pallas as pl
from jax.experimental.pallas import tpu as pltpu
from jax.experimental.pallas import tpu_sc as plsc

Q = 512
K = 100000
D = 768
K_CHUNK = 100
K_DOC = 10
N_DOCS = 10000

KT = 2048
K_PAD = ((K + KT - 1) // KT) * KT
N_TILES = K_PAD // KT
T_TILE = 4

NEG = -3.4e38

NC = 2
NS = 16
NW = NC * NS
LANES = 16
ROWS_PER_W = Q // NW
CAP = 512
CHUNK = K_PAD // 8
N_CHUNKS = 8


def _proj_kernel(q_ref, w_ref, b_ref, out_ref):
    acc = jnp.dot(q_ref[...], w_ref[...], preferred_element_type=jnp.float32)
    out_ref[...] = jnp.tanh(acc + b_ref[...])


def _project(queries, W, b):
    return pl.pallas_call(
        _proj_kernel,
        out_shape=jax.ShapeDtypeStruct((Q, D), jnp.float32),
    )(queries, W, b.reshape(1, D))


def _sim_kernel(q_ref, k_ref, out_ref, t4_ref):
    i = pl.program_id(0)
    sim = lax.dot_general(
        q_ref[...], k_ref[...], (((1,), (1,)), ((), ())),
        preferred_element_type=jnp.float32)
    col = i * KT + lax.broadcasted_iota(jnp.int32, (Q, KT), 1)
    simm = jnp.where(col < K, sim, NEG)
    out_ref[...] = simm
    x = simm
    for t in range(T_TILE):
        m = jnp.max(x, axis=1, keepdims=True)
        t4_ref[0, t, :] = m[:, 0]
        if t + 1 < T_TILE:
            x = jnp.where(x == m, NEG, x)


def _sim(q, keys_pad):
    return pl.pallas_call(
        _sim_kernel,
        grid=(N_TILES,),
        in_specs=[
            pl.BlockSpec((Q, D), lambda i: (0, 0)),
            pl.BlockSpec((KT, D), lambda i: (i, 0)),
        ],
        out_specs=[
            pl.BlockSpec((Q, KT), lambda i: (0, i)),
            pl.BlockSpec((1, T_TILE, Q), lambda i: (i, 0, 0)),
        ],
        out_shape=[
            jax.ShapeDtypeStruct((Q, K_PAD), jnp.float32),
            jax.ShapeDtypeStruct((N_TILES, T_TILE, Q), jnp.float32),
        ],
    )(q, keys_pad)


def _thresh_kernel(t4_ref, u_ref):
    x = t4_ref[...].reshape(N_TILES * T_TILE, Q)
    m = None
    for _ in range(K_CHUNK):
        m = jnp.max(x, axis=0, keepdims=True)
        x = jnp.where(x == m, NEG, x)
    u_ref[...] = m


def _thresh(t4):
    return pl.pallas_call(
        _thresh_kernel,
        out_shape=jax.ShapeDtypeStruct((1, Q), jnp.float32),
    )(t4)


def _filter_body(sim_hbm, u_hbm, val_hbm, idx_hbm, buf, vbuf, ibuf, ubuf):
    wid = lax.axis_index("s") * NC + lax.axis_index("c")
    row0 = wid * ROWS_PER_W
    pltpu.sync_copy(u_hbm.at[pl.ds(row0, ROWS_PER_W)],
                    ubuf.at[pl.ds(0, ROWS_PER_W)])

    def row_body(j, _):
        q = row0 + j
        uj = ubuf[pl.ds(j, LANES)][0]

        def init_body(t, _c):
            vbuf[pl.ds(t * LANES, LANES)] = jnp.full((LANES,), NEG, jnp.float32)
            ibuf[pl.ds(t * LANES, LANES)] = jnp.zeros((LANES,), jnp.int32)
            return _c

        lax.fori_loop(0, (CAP + LANES) // LANES, init_body, 0)

        def chunk_body(c, off):
            pltpu.sync_copy(
                sim_hbm.at[pl.ds(q * K_PAD + c * CHUNK, CHUNK)], buf)
            base = c * CHUNK

            def vec_body(t, off):
                v = buf[pl.ds(t * LANES, LANES)]
                msk = v >= uj
                cnt = plsc.all_reduce_population_count(msk)
                if getattr(cnt, "ndim", 0):
                    cnt = cnt[0]
                mi = jnp.where(msk, 1, 0)

                @pl.when(cnt > 0)
                def _slow():
                    o = off
                    for lane in range(LANES):
                        vbuf[pl.ds(o, LANES)] = jnp.full(
                            (LANES,), v[lane], jnp.float32)
                        ibuf[pl.ds(o, LANES)] = jnp.full(
                            (LANES,), base + t * LANES + lane, jnp.int32)
                        o = o + mi[lane]

                return jnp.minimum(off + cnt, CAP)

            return lax.fori_loop(0, CHUNK // LANES, vec_body, off)

        off_end = lax.fori_loop(0, N_CHUNKS, chunk_body, jnp.int32(0))
        vbuf[pl.ds(off_end, LANES)] = jnp.full((LANES,), NEG, jnp.float32)
        ibuf[pl.ds(off_end, LANES)] = jnp.zeros((LANES,), jnp.int32)
        pltpu.sync_copy(vbuf.at[pl.ds(0, CAP)], val_hbm.at[q])
        pltpu.sync_copy(ibuf.at[pl.ds(0, CAP)], idx_hbm.at[q])
        return _

    lax.fori_loop(0, ROWS_PER_W, row_body, 0)


_filter = functools.partial(
    pl.kernel,
    mesh=plsc.VectorSubcoreMesh(core_axis_name="c", subcore_axis_name="s"),
    compiler_params=pltpu.CompilerParams(needs_layout_passes=False),
    out_type=[
        jax.ShapeDtypeStruct((Q, CAP), jnp.float32),
        jax.ShapeDtypeStruct((Q, CAP), jnp.int32),
    ],
    scratch_types=[
        pltpu.VMEM((CHUNK,), jnp.float32),
        pltpu.VMEM((CAP + LANES,), jnp.float32),
        pltpu.VMEM((CAP + LANES,), jnp.int32),
        pltpu.VMEM((ROWS_PER_W + LANES,), jnp.float32),
    ],
)(_filter_body)


def kernel(queries, keys, doc_ids, W, b):
    q = _project(queries, W, b)
    keys_pad = jnp.pad(keys, ((0, K_PAD - K), (0, 0)))
    sim, t4 = _sim(q, keys_pad)
    u = _thresh(t4).reshape(Q)
    cvals, cidx = _filter(sim.reshape(Q * K_PAD), u)
    tv, tpos = lax.top_k(cvals, K_CHUNK)
    inds = jnp.take_along_axis(cidx, tpos, axis=1)
    chunks = jnp.take(keys, inds, axis=0)
    qn = q / (jnp.linalg.norm(q, axis=-1, keepdims=True) + 1e-8)
    cn = chunks / (jnp.linalg.norm(chunks, axis=-1, keepdims=True) + 1e-8)
    cos = jnp.sum(qn[:, None, :] * cn, axis=-1)
    scores = jnp.exp(cos)
    docs = jnp.take(doc_ids, inds, axis=0)
    row = jnp.arange(Q, dtype=jnp.int32)[:, None]
    doc_scores = jnp.zeros((Q, N_DOCS), dtype=jnp.float32)
    doc_scores = doc_scores.at[row, docs].add(scores)
    top_scores, top_docs = lax.top_k(doc_scores, K_DOC)
    return top_scores, top_docs


# speedup vs baseline: 6.2592x; 1.6817x over previous
import functools

import jax
import jax.numpy as jnp
from jax import lax
from jax.experimental import pallas as pl
from jax.experimental.pallas import tpu as pltpu
from jax.experimental.pallas import tpu_sc as plsc

Q = 512
K = 100000
D = 768
K_CHUNK = 100
K_DOC = 10
N_DOCS = 10000

KT = 2048
K_PAD = ((K + KT - 1) // KT) * KT
N_TILES = K_PAD // KT
T_TILE = 4

NEG = -3.4e38

NC = 2
NS = 16
NW = NC * NS
LANES = 16
ROWS_PER_W = Q // NW
CAP = 512
CHUNK = K_PAD // 8
N_CHUNKS = 8
GRP = 256


def _proj_kernel(q_ref, w_ref, b_ref, out_ref):
    acc = jnp.dot(q_ref[...], w_ref[...], preferred_element_type=jnp.float32)
    out_ref[...] = jnp.tanh(acc + b_ref[...])


def _project(queries, W, b):
    return pl.pallas_call(
        _proj_kernel,
        out_shape=jax.ShapeDtypeStruct((Q, D), jnp.float32),
    )(queries, W, b.reshape(1, D))


def _sim_kernel(q_ref, k_ref, out_ref, t4_ref):
    i = pl.program_id(0)
    sim = lax.dot_general(
        q_ref[...], k_ref[...], (((1,), (1,)), ((), ())),
        preferred_element_type=jnp.float32)
    col = i * KT + lax.broadcasted_iota(jnp.int32, (Q, KT), 1)
    simm = jnp.where(col < K, sim, NEG)
    out_ref[...] = simm
    x = simm
    for t in range(T_TILE):
        m = jnp.max(x, axis=1, keepdims=True)
        t4_ref[0, t, :] = m[:, 0]
        if t + 1 < T_TILE:
            x = jnp.where(x == m, NEG, x)


def _sim(q, keys_pad):
    return pl.pallas_call(
        _sim_kernel,
        grid=(N_TILES,),
        in_specs=[
            pl.BlockSpec((Q, D), lambda i: (0, 0)),
            pl.BlockSpec((KT, D), lambda i: (i, 0)),
        ],
        out_specs=[
            pl.BlockSpec((Q, KT), lambda i: (0, i)),
            pl.BlockSpec((1, T_TILE, Q), lambda i: (i, 0, 0)),
        ],
        out_shape=[
            jax.ShapeDtypeStruct((Q, K_PAD), jnp.float32),
            jax.ShapeDtypeStruct((N_TILES, T_TILE, Q), jnp.float32),
        ],
    )(q, keys_pad)


def _thresh_kernel(t4_ref, u_ref):
    x = t4_ref[...].reshape(N_TILES * T_TILE, Q)
    m = None
    for _ in range(K_CHUNK):
        m = jnp.max(x, axis=0, keepdims=True)
        x = jnp.where(x == m, NEG, x)
    u_ref[...] = m


def _thresh(t4):
    return pl.pallas_call(
        _thresh_kernel,
        out_shape=jax.ShapeDtypeStruct((1, Q), jnp.float32),
    )(t4)


N_DOCS_PAD = 10112


def _top10_kernel(ds_ref, s_ref, d_ref):
    x = ds_ref[...]
    colidx = lax.broadcasted_iota(jnp.int32, (Q, N_DOCS_PAD), 1)
    for t in range(K_DOC):
        m = jnp.max(x, axis=1, keepdims=True)
        ic = jnp.min(jnp.where(x == m, colidx, jnp.int32(2**30)),
                     axis=1, keepdims=True)
        s_ref[t, :] = m[:, 0]
        d_ref[t, :] = ic[:, 0]
        if t + 1 < K_DOC:
            x = jnp.where(colidx == ic, NEG, x)


def _top10(doc_scores):
    return pl.pallas_call(
        _top10_kernel,
        out_shape=[
            jax.ShapeDtypeStruct((K_DOC, Q), jnp.float32),
            jax.ShapeDtypeStruct((K_DOC, Q), jnp.int32),
        ],
    )(doc_scores)


def _filter_body(sim_hbm, u_hbm, val_hbm, idx_hbm, buf, vbuf, ibuf, ubuf):
    wid = lax.axis_index("s") * NC + lax.axis_index("c")
    row0 = wid * ROWS_PER_W
    pltpu.sync_copy(u_hbm.at[pl.ds(row0, ROWS_PER_W)],
                    ubuf.at[pl.ds(0, ROWS_PER_W)])

    def row_body(j, _):
        q = row0 + j
        uj = ubuf[pl.ds(j, LANES)][0]

        def init_body(t, _c):
            vbuf[pl.ds(t * LANES, LANES)] = jnp.full((LANES,), NEG, jnp.float32)
            ibuf[pl.ds(t * LANES, LANES)] = jnp.zeros((LANES,), jnp.int32)
            return _c

        lax.fori_loop(0, (CAP + LANES) // LANES, init_body, 0)

        def chunk_body(c, off):
            pltpu.sync_copy(
                sim_hbm.at[pl.ds(q * K_PAD + c * CHUNK, CHUNK)], buf)
            base = c * CHUNK

            def grp_body(g, off):
                gb = g * GRP
                macc = buf[pl.ds(gb, LANES)]
                for k in range(1, GRP // LANES):
                    macc = jnp.maximum(macc, buf[pl.ds(gb + k * LANES, LANES)])
                hit = plsc.all_reduce_population_count(macc >= uj)
                if getattr(hit, "ndim", 0):
                    hit = hit[0]

                def slow(o):
                    def vec_body(t, o2):
                        eb = gb + t * LANES
                        v = buf[pl.ds(eb, LANES)]
                        msk = v >= uj
                        cnt = plsc.all_reduce_population_count(msk)
                        if getattr(cnt, "ndim", 0):
                            cnt = cnt[0]
                        mi = jnp.where(msk, 1, 0)

                        @pl.when(cnt > 0)
                        def _store():
                            o3 = o2
                            for lane in range(LANES):
                                vbuf[pl.ds(o3, LANES)] = jnp.full(
                                    (LANES,), v[lane], jnp.float32)
                                ibuf[pl.ds(o3, LANES)] = jnp.full(
                                    (LANES,), base + eb + lane, jnp.int32)
                                o3 = o3 + mi[lane]

                        return jnp.minimum(o2 + cnt, CAP)

                    return lax.fori_loop(0, GRP // LANES, vec_body, o)

                return lax.cond(hit > 0, slow, lambda o: o, off)

            return lax.fori_loop(0, CHUNK // GRP, grp_body, off)

        off_end = lax.fori_loop(0, N_CHUNKS, chunk_body, jnp.int32(0))
        vbuf[pl.ds(off_end, LANES)] = jnp.full((LANES,), NEG, jnp.float32)
        ibuf[pl.ds(off_end, LANES)] = jnp.zeros((LANES,), jnp.int32)
        pltpu.sync_copy(vbuf.at[pl.ds(0, CAP)], val_hbm.at[q])
        pltpu.sync_copy(ibuf.at[pl.ds(0, CAP)], idx_hbm.at[q])
        return _

    lax.fori_loop(0, ROWS_PER_W, row_body, 0)


_filter = functools.partial(
    pl.kernel,
    mesh=plsc.VectorSubcoreMesh(core_axis_name="c", subcore_axis_name="s"),
    compiler_params=pltpu.CompilerParams(needs_layout_passes=False),
    out_type=[
        jax.ShapeDtypeStruct((Q, CAP), jnp.float32),
        jax.ShapeDtypeStruct((Q, CAP), jnp.int32),
    ],
    scratch_types=[
        pltpu.VMEM((CHUNK,), jnp.float32),
        pltpu.VMEM((CAP + LANES,), jnp.float32),
        pltpu.VMEM((CAP + LANES,), jnp.int32),
        pltpu.VMEM((ROWS_PER_W + LANES,), jnp.float32),
    ],
)(_filter_body)


def kernel(queries, keys, doc_ids, W, b):
    q = _project(queries, W, b)
    keys_pad = jnp.pad(keys, ((0, K_PAD - K), (0, 0)))
    sim, t4 = _sim(q, keys_pad)
    u = _thresh(t4).reshape(Q)
    cvals, cidx = _filter(sim.reshape(Q * K_PAD), u)
    tv, tpos = lax.top_k(cvals, K_CHUNK)
    inds = jnp.take_along_axis(cidx, tpos, axis=1)
    chunks = jnp.take(keys, inds, axis=0)
    qn = q / (jnp.linalg.norm(q, axis=-1, keepdims=True) + 1e-8)
    cn = chunks / (jnp.linalg.norm(chunks, axis=-1, keepdims=True) + 1e-8)
    cos = jnp.sum(qn[:, None, :] * cn, axis=-1)
    scores = jnp.exp(cos)
    docs = jnp.take(doc_ids, inds, axis=0)
    row = jnp.arange(Q, dtype=jnp.int32)[:, None]
    doc_scores = jnp.zeros((Q, N_DOCS_PAD), dtype=jnp.float32)
    doc_scores = doc_scores.at[row, docs].add(scores)
    s_t, d_t = _top10(doc_scores)
    return s_t.T, d_t.T


# speedup vs baseline: 6.7965x; 1.0858x over previous
import functools

import jax
import jax.numpy as jnp
from jax import lax
from jax.experimental import pallas as pl
from jax.experimental.pallas import tpu as pltpu
from jax.experimental.pallas import tpu_sc as plsc

Q = 512
K = 100000
D = 768
K_CHUNK = 100
K_DOC = 10
N_DOCS = 10000

KT = 2048
K_PAD = ((K + KT - 1) // KT) * KT
N_TILES = K_PAD // KT
T_TILE = 4

NEG = -3.4e38

NC = 2
NS = 16
NW = NC * NS
LANES = 16
ROWS_PER_W = Q // NW
CAP = 512
CHUNK = K_PAD // 8
N_CHUNKS = 8
GRP = 256


def _proj_kernel(q_ref, w_ref, b_ref, out_ref):
    acc = jnp.dot(q_ref[...], w_ref[...], preferred_element_type=jnp.float32)
    out_ref[...] = jnp.tanh(acc + b_ref[...])


def _project(queries, W, b):
    return pl.pallas_call(
        _proj_kernel,
        out_shape=jax.ShapeDtypeStruct((Q, D), jnp.float32),
    )(queries, W, b.reshape(1, D))


def _sim_kernel(q_ref, k_ref, out_ref, t4_ref):
    i = pl.program_id(0)
    sim = lax.dot_general(
        q_ref[...], k_ref[...], (((1,), (1,)), ((), ())),
        preferred_element_type=jnp.float32)
    col = i * KT + lax.broadcasted_iota(jnp.int32, (Q, KT), 1)
    simm = jnp.where(col < K, sim, NEG)
    out_ref[...] = simm
    x = simm
    for t in range(T_TILE):
        m = jnp.max(x, axis=1, keepdims=True)
        t4_ref[0, t, :] = m[:, 0]
        if t + 1 < T_TILE:
            x = jnp.where(x == m, NEG, x)


def _sim(q, keys_pad):
    return pl.pallas_call(
        _sim_kernel,
        grid=(N_TILES,),
        in_specs=[
            pl.BlockSpec((Q, D), lambda i: (0, 0)),
            pl.BlockSpec((KT, D), lambda i: (i, 0)),
        ],
        out_specs=[
            pl.BlockSpec((Q, KT), lambda i: (0, i)),
            pl.BlockSpec((1, T_TILE, Q), lambda i: (i, 0, 0)),
        ],
        out_shape=[
            jax.ShapeDtypeStruct((Q, K_PAD), jnp.float32),
            jax.ShapeDtypeStruct((N_TILES, T_TILE, Q), jnp.float32),
        ],
    )(q, keys_pad)


def _thresh_kernel(t4_ref, u_ref):
    x = t4_ref[...].reshape(N_TILES * T_TILE, Q)
    m = None
    for _ in range(K_CHUNK):
        m = jnp.max(x, axis=0, keepdims=True)
        x = jnp.where(x == m, NEG, x)
    u_ref[...] = m


def _thresh(t4):
    return pl.pallas_call(
        _thresh_kernel,
        out_shape=jax.ShapeDtypeStruct((1, Q), jnp.float32),
    )(t4)


N_DOCS_PAD = 10112


def _top100_kernel(cv_ref, p_ref):
    x = cv_ref[...]
    colidx = lax.broadcasted_iota(jnp.int32, (Q, CAP), 1)
    for t in range(K_CHUNK):
        m = jnp.max(x, axis=1, keepdims=True)
        ic = jnp.min(jnp.where(x == m, colidx, jnp.int32(2**30)),
                     axis=1, keepdims=True)
        p_ref[t, :] = ic[:, 0]
        if t + 1 < K_CHUNK:
            x = jnp.where(colidx == ic, NEG, x)


def _top100(cvals):
    return pl.pallas_call(
        _top100_kernel,
        out_shape=jax.ShapeDtypeStruct((K_CHUNK, Q), jnp.int32),
    )(cvals)


def _top10_kernel(ds_ref, s_ref, d_ref):
    x = ds_ref[...]
    colidx = lax.broadcasted_iota(jnp.int32, (Q, N_DOCS_PAD), 1)
    for t in range(K_DOC):
        m = jnp.max(x, axis=1, keepdims=True)
        ic = jnp.min(jnp.where(x == m, colidx, jnp.int32(2**30)),
                     axis=1, keepdims=True)
        s_ref[t, :] = m[:, 0]
        d_ref[t, :] = ic[:, 0]
        if t + 1 < K_DOC:
            x = jnp.where(colidx == ic, NEG, x)


def _top10(doc_scores):
    return pl.pallas_call(
        _top10_kernel,
        out_shape=[
            jax.ShapeDtypeStruct((K_DOC, Q), jnp.float32),
            jax.ShapeDtypeStruct((K_DOC, Q), jnp.int32),
        ],
    )(doc_scores)


def _filter_body(sim_hbm, u_hbm, val_hbm, idx_hbm, buf, vbuf, ibuf, ubuf):
    wid = lax.axis_index("s") * NC + lax.axis_index("c")
    row0 = wid * ROWS_PER_W
    pltpu.sync_copy(u_hbm.at[pl.ds(row0, ROWS_PER_W)],
                    ubuf.at[pl.ds(0, ROWS_PER_W)])

    def row_body(j, _):
        q = row0 + j
        uj = ubuf[pl.ds(j, LANES)][0]

        def init_body(t, _c):
            vbuf[pl.ds(t * LANES, LANES)] = jnp.full((LANES,), NEG, jnp.float32)
            ibuf[pl.ds(t * LANES, LANES)] = jnp.zeros((LANES,), jnp.int32)
            return _c

        lax.fori_loop(0, (CAP + LANES) // LANES, init_body, 0)

        def chunk_body(c, off):
            pltpu.sync_copy(
                sim_hbm.at[pl.ds(q * K_PAD + c * CHUNK, CHUNK)], buf)
            base = c * CHUNK

            def grp_body(g, off):
                gb = g * GRP
                macc = buf[pl.ds(gb, LANES)]
                for k in range(1, GRP // LANES):
                    macc = jnp.maximum(macc, buf[pl.ds(gb + k * LANES, LANES)])
                hit = plsc.all_reduce_population_count(macc >= uj)
                if getattr(hit, "ndim", 0):
                    hit = hit[0]

                def slow(o):
                    def vec_body(t, o2):
                        eb = gb + t * LANES
                        v = buf[pl.ds(eb, LANES)]
                        msk = v >= uj
                        cnt = plsc.all_reduce_population_count(msk)
                        if getattr(cnt, "ndim", 0):
                            cnt = cnt[0]
                        mi = jnp.where(msk, 1, 0)

                        @pl.when(cnt > 0)
                        def _store():
                            o3 = o2
                            for lane in range(LANES):
                                vbuf[pl.ds(o3, LANES)] = jnp.full(
                                    (LANES,), v[lane], jnp.float32)
                                ibuf[pl.ds(o3, LANES)] = jnp.full(
                                    (LANES,), base + eb + lane, jnp.int32)
                                o3 = o3 + mi[lane]

                        return jnp.minimum(o2 + cnt, CAP)

                    return lax.fori_loop(0, GRP // LANES, vec_body, o)

                return lax.cond(hit > 0, slow, lambda o: o, off)

            return lax.fori_loop(0, CHUNK // GRP, grp_body, off)

        off_end = lax.fori_loop(0, N_CHUNKS, chunk_body, jnp.int32(0))
        vbuf[pl.ds(off_end, LANES)] = jnp.full((LANES,), NEG, jnp.float32)
        ibuf[pl.ds(off_end, LANES)] = jnp.zeros((LANES,), jnp.int32)
        pltpu.sync_copy(vbuf.at[pl.ds(0, CAP)], val_hbm.at[q])
        pltpu.sync_copy(ibuf.at[pl.ds(0, CAP)], idx_hbm.at[q])
        return _

    lax.fori_loop(0, ROWS_PER_W, row_body, 0)


_filter = functools.partial(
    pl.kernel,
    mesh=plsc.VectorSubcoreMesh(core_axis_name="c", subcore_axis_name="s"),
    compiler_params=pltpu.CompilerParams(needs_layout_passes=False),
    out_type=[
        jax.ShapeDtypeStruct((Q, CAP), jnp.float32),
        jax.ShapeDtypeStruct((Q, CAP), jnp.int32),
    ],
    scratch_types=[
        pltpu.VMEM((CHUNK,), jnp.float32),
        pltpu.VMEM((CAP + LANES,), jnp.float32),
        pltpu.VMEM((CAP + LANES,), jnp.int32),
        pltpu.VMEM((ROWS_PER_W + LANES,), jnp.float32),
    ],
)(_filter_body)


def kernel(queries, keys, doc_ids, W, b):
    q = _project(queries, W, b)
    sim, t4 = _sim(q, keys)
    u = _thresh(t4).reshape(Q)
    cvals, cidx = _filter(sim.reshape(Q * K_PAD), u)
    tpos = _top100(cvals).T
    inds = jnp.take_along_axis(cidx, tpos, axis=1)
    chunks = jnp.take(keys, inds, axis=0)
    qn = q / (jnp.linalg.norm(q, axis=-1, keepdims=True) + 1e-8)
    cn = chunks / (jnp.linalg.norm(chunks, axis=-1, keepdims=True) + 1e-8)
    cos = jnp.sum(qn[:, None, :] * cn, axis=-1)
    scores = jnp.exp(cos)
    docs = jnp.take(doc_ids, inds, axis=0)
    row = jnp.arange(Q, dtype=jnp.int32)[:, None]
    doc_scores = jnp.zeros((Q, N_DOCS_PAD), dtype=jnp.float32)
    doc_scores = doc_scores.at[row, docs].add(scores)
    s_t, d_t = _top10(doc_scores)
    return s_t.T, d_t.T


# speedup vs baseline: 7.2828x; 1.0716x over previous
import functools

import jax
import jax.numpy as jnp
from jax import lax
from jax.experimental import pallas as pl
from jax.experimental.pallas import tpu as pltpu
from jax.experimental.pallas import tpu_sc as plsc

Q = 512
K = 100000
D = 768
K_CHUNK = 100
K_DOC = 10
N_DOCS = 10000

KT = 2048
K_PAD = ((K + KT - 1) // KT) * KT
N_TILES = K_PAD // KT
T_TILE = 4

NEG = -3.4e38

NC = 2
NS = 16
NW = NC * NS
LANES = 16
ROWS_PER_W = Q // NW
CAP = 512
CHUNK = K_PAD // 8
N_CHUNKS = 8
GRP = 256


def _proj_kernel(q_ref, w_ref, b_ref, out_ref):
    acc = jnp.dot(q_ref[...], w_ref[...], preferred_element_type=jnp.float32)
    out_ref[...] = jnp.tanh(acc + b_ref[...])


def _project(queries, W, b):
    return pl.pallas_call(
        _proj_kernel,
        out_shape=jax.ShapeDtypeStruct((Q, D), jnp.float32),
    )(queries, W, b.reshape(1, D))


def _sim_kernel(q_ref, k_ref, out_ref, t4_ref):
    i = pl.program_id(0)
    sim = lax.dot_general(
        q_ref[...], k_ref[...], (((1,), (1,)), ((), ())),
        preferred_element_type=jnp.float32)
    col = i * KT + lax.broadcasted_iota(jnp.int32, (Q, KT), 1)
    simm = jnp.where(col < K, sim, NEG)
    out_ref[...] = simm
    x = simm
    for t in range(T_TILE):
        m = jnp.max(x, axis=1, keepdims=True)
        t4_ref[0, t, :] = m[:, 0]
        if t + 1 < T_TILE:
            x = jnp.where(x == m, NEG, x)


def _sim(q, keys_pad):
    return pl.pallas_call(
        _sim_kernel,
        grid=(N_TILES,),
        in_specs=[
            pl.BlockSpec((Q, D), lambda i: (0, 0)),
            pl.BlockSpec((KT, D), lambda i: (i, 0)),
        ],
        out_specs=[
            pl.BlockSpec((Q, KT), lambda i: (0, i)),
            pl.BlockSpec((1, T_TILE, Q), lambda i: (i, 0, 0)),
        ],
        out_shape=[
            jax.ShapeDtypeStruct((Q, K_PAD), jnp.float32),
            jax.ShapeDtypeStruct((N_TILES, T_TILE, Q), jnp.float32),
        ],
    )(q, keys_pad)


def _thresh_kernel(t4_ref, u_ref):
    x = t4_ref[...].reshape(N_TILES * T_TILE, Q)
    m = None
    for _ in range(K_CHUNK):
        m = jnp.max(x, axis=0, keepdims=True)
        x = jnp.where(x == m, NEG, x)
    u_ref[...] = m


def _thresh(t4):
    return pl.pallas_call(
        _thresh_kernel,
        out_shape=jax.ShapeDtypeStruct((1, Q), jnp.float32),
    )(t4)


N_DOCS_PAD = 10112


def _top100_kernel(cv_ref, p_ref):
    x = cv_ref[...]
    colidx = lax.broadcasted_iota(jnp.int32, (Q, CAP), 1)
    for t in range(K_CHUNK):
        m = jnp.max(x, axis=1, keepdims=True)
        ic = jnp.min(jnp.where(x == m, colidx, jnp.int32(2**30)),
                     axis=1, keepdims=True)
        p_ref[t, :] = ic[:, 0]
        if t + 1 < K_CHUNK:
            x = jnp.where(colidx == ic, NEG, x)


def _top100(cvals):
    return pl.pallas_call(
        _top100_kernel,
        out_shape=jax.ShapeDtypeStruct((K_CHUNK, Q), jnp.int32),
    )(cvals)


def _top10_kernel(ds_ref, s_ref, d_ref):
    x = ds_ref[...]
    colidx = lax.broadcasted_iota(jnp.int32, (Q, N_DOCS_PAD), 1)
    for t in range(K_DOC):
        m = jnp.max(x, axis=1, keepdims=True)
        ic = jnp.min(jnp.where(x == m, colidx, jnp.int32(2**30)),
                     axis=1, keepdims=True)
        s_ref[t, :] = m[:, 0]
        d_ref[t, :] = ic[:, 0]
        if t + 1 < K_DOC:
            x = jnp.where(colidx == ic, NEG, x)


def _top10(doc_scores):
    return pl.pallas_call(
        _top10_kernel,
        out_shape=[
            jax.ShapeDtypeStruct((K_DOC, Q), jnp.float32),
            jax.ShapeDtypeStruct((K_DOC, Q), jnp.int32),
        ],
    )(doc_scores)


def _filter_body(sim_hbm, u_hbm, val_hbm, idx_hbm, buf, vbuf, ibuf, ubuf):
    wid = lax.axis_index("s") * NC + lax.axis_index("c")
    row0 = wid * ROWS_PER_W
    pltpu.sync_copy(u_hbm.at[pl.ds(row0, ROWS_PER_W)],
                    ubuf.at[pl.ds(0, ROWS_PER_W)])

    def row_body(j, _):
        q = row0 + j
        uj = ubuf[pl.ds(j, LANES)][0]

        def init_body(t, _c):
            vbuf[pl.ds(t * LANES, LANES)] = jnp.full((LANES,), NEG, jnp.float32)
            ibuf[pl.ds(t * LANES, LANES)] = jnp.zeros((LANES,), jnp.int32)
            return _c

        lax.fori_loop(0, (CAP + LANES) // LANES, init_body, 0)

        def chunk_body(c, off):
            pltpu.sync_copy(
                sim_hbm.at[q, pl.ds(c * CHUNK, CHUNK)], buf)
            base = c * CHUNK

            def grp_body(g, off):
                gb = g * GRP
                macc = buf[pl.ds(gb, LANES)]
                for k in range(1, GRP // LANES):
                    macc = jnp.maximum(macc, buf[pl.ds(gb + k * LANES, LANES)])
                hit = plsc.all_reduce_population_count(macc >= uj)
                if getattr(hit, "ndim", 0):
                    hit = hit[0]

                def slow(o):
                    def vec_body(t, o2):
                        eb = gb + t * LANES
                        v = buf[pl.ds(eb, LANES)]
                        msk = v >= uj
                        cnt = plsc.all_reduce_population_count(msk)
                        if getattr(cnt, "ndim", 0):
                            cnt = cnt[0]
                        mi = jnp.where(msk, 1, 0)

                        @pl.when(cnt > 0)
                        def _store():
                            o3 = o2
                            for lane in range(LANES):
                                vbuf[pl.ds(o3, LANES)] = jnp.full(
                                    (LANES,), v[lane], jnp.float32)
                                ibuf[pl.ds(o3, LANES)] = jnp.full(
                                    (LANES,), base + eb + lane, jnp.int32)
                                o3 = o3 + mi[lane]

                        return jnp.minimum(o2 + cnt, CAP)

                    return lax.fori_loop(0, GRP // LANES, vec_body, o)

                return lax.cond(hit > 0, slow, lambda o: o, off)

            return lax.fori_loop(0, CHUNK // GRP, grp_body, off)

        off_end = lax.fori_loop(0, N_CHUNKS, chunk_body, jnp.int32(0))
        vbuf[pl.ds(off_end, LANES)] = jnp.full((LANES,), NEG, jnp.float32)
        ibuf[pl.ds(off_end, LANES)] = jnp.zeros((LANES,), jnp.int32)
        pltpu.sync_copy(vbuf.at[pl.ds(0, CAP)], val_hbm.at[q])
        pltpu.sync_copy(ibuf.at[pl.ds(0, CAP)], idx_hbm.at[q])
        return _

    lax.fori_loop(0, ROWS_PER_W, row_body, 0)


_filter = functools.partial(
    pl.kernel,
    mesh=plsc.VectorSubcoreMesh(core_axis_name="c", subcore_axis_name="s"),
    compiler_params=pltpu.CompilerParams(needs_layout_passes=False),
    out_type=[
        jax.ShapeDtypeStruct((Q, CAP), jnp.float32),
        jax.ShapeDtypeStruct((Q, CAP), jnp.int32),
    ],
    scratch_types=[
        pltpu.VMEM((CHUNK,), jnp.float32),
        pltpu.VMEM((CAP + LANES,), jnp.float32),
        pltpu.VMEM((CAP + LANES,), jnp.int32),
        pltpu.VMEM((ROWS_PER_W + LANES,), jnp.float32),
    ],
)(_filter_body)


def kernel(queries, keys, doc_ids, W, b):
    q = _project(queries, W, b)
    sim, t4 = _sim(q, keys)
    u = _thresh(t4).reshape(Q)
    cvals, cidx = _filter(sim, u)
    tpos = _top100(cvals).T
    inds = jnp.take_along_axis(cidx, tpos, axis=1)
    chunks = jnp.take(keys, inds, axis=0)
    qn = q / (jnp.linalg.norm(q, axis=-1, keepdims=True) + 1e-8)
    cn = chunks / (jnp.linalg.norm(chunks, axis=-1, keepdims=True) + 1e-8)
    cos = jnp.sum(qn[:, None, :] * cn, axis=-1)
    scores = jnp.exp(cos)
    docs = jnp.take(doc_ids, inds, axis=0)
    row = jnp.arange(Q, dtype=jnp.int32)[:, None]
    doc_scores = jnp.zeros((Q, N_DOCS_PAD), dtype=jnp.float32)
    doc_scores = doc_scores.at[row, docs].add(scores)
    s_t, d_t = _top10(doc_scores)
    return s_t.T, d_t.T


# speedup vs baseline: 7.3015x; 1.0026x over previous
import functools

import jax
import jax.numpy as jnp
from jax import lax
from jax.experimental import pallas as pl
from jax.experimental.pallas import tpu as pltpu
from jax.experimental.pallas import tpu_sc as plsc

Q = 512
K = 100000
D = 768
K_CHUNK = 100
K_DOC = 10
N_DOCS = 10000

KT = 2048
K_PAD = ((K + KT - 1) // KT) * KT
N_TILES = K_PAD // KT
T_TILE = 4

NEG = -3.4e38

NC = 2
NS = 16
NW = NC * NS
LANES = 16
ROWS_PER_W = Q // NW
CAP = 512
CHUNK = K_PAD // 8
N_CHUNKS = 8
GRP = 256


def _proj_kernel(q_ref, w_ref, b_ref, out_ref):
    acc = jnp.dot(q_ref[...], w_ref[...], preferred_element_type=jnp.float32)
    out_ref[...] = jnp.tanh(acc + b_ref[...])


def _project(queries, W, b):
    return pl.pallas_call(
        _proj_kernel,
        out_shape=jax.ShapeDtypeStruct((Q, D), jnp.float32),
    )(queries, W, b.reshape(1, D))


def _sim_kernel(q_ref, k_ref, out_ref, t4_ref):
    i = pl.program_id(0)
    sim = lax.dot_general(
        q_ref[...], k_ref[...], (((1,), (1,)), ((), ())),
        preferred_element_type=jnp.float32)
    col = i * KT + lax.broadcasted_iota(jnp.int32, (Q, KT), 1)
    simm = jnp.where(col < K, sim, NEG)
    out_ref[...] = simm
    x = simm
    for t in range(T_TILE):
        m = jnp.max(x, axis=1, keepdims=True)
        t4_ref[0, t, :] = m[:, 0]
        if t + 1 < T_TILE:
            x = jnp.where(x == m, NEG, x)


def _sim(q, keys_arr):
    return pl.pallas_call(
        _sim_kernel,
        grid=(N_TILES,),
        in_specs=[
            pl.BlockSpec((Q, D), lambda i: (0, 0)),
            pl.BlockSpec((KT, D), lambda i: (i, 0)),
        ],
        out_specs=[
            pl.BlockSpec((Q, KT), lambda i: (0, i)),
            pl.BlockSpec((1, T_TILE, Q), lambda i: (i, 0, 0)),
        ],
        out_shape=[
            jax.ShapeDtypeStruct((Q, K_PAD), jnp.float32),
            jax.ShapeDtypeStruct((N_TILES, T_TILE, Q), jnp.float32),
        ],
    )(q, keys_arr)


def _thresh_kernel(t4_ref, u_ref):
    x = t4_ref[...].reshape(N_TILES * T_TILE, Q)
    m = None
    for _ in range(K_CHUNK):
        m = jnp.max(x, axis=0, keepdims=True)
        x = jnp.where(x == m, NEG, x)
    u_ref[...] = m


def _thresh(t4):
    return pl.pallas_call(
        _thresh_kernel,
        out_shape=jax.ShapeDtypeStruct((1, Q), jnp.float32),
    )(t4)


N_DOCS_PAD = 10112


def _top100_kernel(cv_ref, p_ref):
    x = cv_ref[...]
    colidx = lax.broadcasted_iota(jnp.int32, (Q, CAP), 1)
    for t in range(K_CHUNK):
        m = jnp.max(x, axis=1, keepdims=True)
        ic = jnp.min(jnp.where(x == m, colidx, jnp.int32(2**30)),
                     axis=1, keepdims=True)
        p_ref[t, :] = ic[:, 0]
        if t + 1 < K_CHUNK:
            x = jnp.where(colidx == ic, NEG, x)


def _top100(cvals):
    return pl.pallas_call(
        _top100_kernel,
        out_shape=jax.ShapeDtypeStruct((K_CHUNK, Q), jnp.int32),
    )(cvals)


def _top10_kernel(ds_ref, s_ref, d_ref):
    x = ds_ref[...]
    colidx = lax.broadcasted_iota(jnp.int32, (Q, N_DOCS_PAD), 1)
    for t in range(K_DOC):
        m = jnp.max(x, axis=1, keepdims=True)
        ic = jnp.min(jnp.where(x == m, colidx, jnp.int32(2**30)),
                     axis=1, keepdims=True)
        s_ref[t, :] = m[:, 0]
        d_ref[t, :] = ic[:, 0]
        if t + 1 < K_DOC:
            x = jnp.where(colidx == ic, NEG, x)


def _top10(doc_scores):
    return pl.pallas_call(
        _top10_kernel,
        out_shape=[
            jax.ShapeDtypeStruct((K_DOC, Q), jnp.float32),
            jax.ShapeDtypeStruct((K_DOC, Q), jnp.int32),
        ],
    )(doc_scores)


def _filter_body(sim_hbm, u_hbm, val_hbm, idx_hbm, buf, vbuf, ibuf, ubuf):
    wid = lax.axis_index("s") * NC + lax.axis_index("c")
    row0 = wid * ROWS_PER_W
    pltpu.sync_copy(u_hbm.at[pl.ds(row0, ROWS_PER_W)],
                    ubuf.at[pl.ds(0, ROWS_PER_W)])

    def row_body(j, _):
        q = row0 + j
        uj = ubuf[pl.ds(j, LANES)][0]

        def init_body(t, _c):
            vbuf[pl.ds(t * LANES, LANES)] = jnp.full((LANES,), NEG, jnp.float32)
            ibuf[pl.ds(t * LANES, LANES)] = jnp.zeros((LANES,), jnp.int32)
            return _c

        lax.fori_loop(0, (CAP + LANES) // LANES, init_body, 0)

        def chunk_body(c, off):
            pltpu.sync_copy(
                sim_hbm.at[q, pl.ds(c * CHUNK, CHUNK)], buf)
            base = c * CHUNK

            def grp_body(g, off):
                gb = g * GRP
                macc = buf[pl.ds(gb, LANES)]
                for k in range(1, GRP // LANES):
                    macc = jnp.maximum(macc, buf[pl.ds(gb + k * LANES, LANES)])
                hit = plsc.all_reduce_population_count(macc >= uj)
                if getattr(hit, "ndim", 0):
                    hit = hit[0]

                def slow(o):
                    def vec_body(t, o2):
                        eb = gb + t * LANES
                        v = buf[pl.ds(eb, LANES)]
                        msk = v >= uj
                        cnt = plsc.all_reduce_population_count(msk)
                        if getattr(cnt, "ndim", 0):
                            cnt = cnt[0]
                        mi = jnp.where(msk, 1, 0)

                        @pl.when(cnt > 0)
                        def _store():
                            o3 = o2
                            for lane in range(LANES):
                                vbuf[pl.ds(o3, LANES)] = jnp.full(
                                    (LANES,), v[lane], jnp.float32)
                                ibuf[pl.ds(o3, LANES)] = jnp.full(
                                    (LANES,), base + eb + lane, jnp.int32)
                                o3 = o3 + mi[lane]

                        return jnp.minimum(o2 + cnt, CAP)

                    return lax.fori_loop(0, GRP // LANES, vec_body, o)

                return lax.cond(hit > 0, slow, lambda o: o, off)

            return lax.fori_loop(0, CHUNK // GRP, grp_body, off)

        off_end = lax.fori_loop(0, N_CHUNKS, chunk_body, jnp.int32(0))
        vbuf[pl.ds(off_end, LANES)] = jnp.full((LANES,), NEG, jnp.float32)
        ibuf[pl.ds(off_end, LANES)] = jnp.zeros((LANES,), jnp.int32)
        pltpu.sync_copy(vbuf.at[pl.ds(0, CAP)], val_hbm.at[q])
        pltpu.sync_copy(ibuf.at[pl.ds(0, CAP)], idx_hbm.at[q])
        return _

    lax.fori_loop(0, ROWS_PER_W, row_body, 0)


_filter = functools.partial(
    pl.kernel,
    mesh=plsc.VectorSubcoreMesh(core_axis_name="c", subcore_axis_name="s"),
    compiler_params=pltpu.CompilerParams(needs_layout_passes=False),
    out_type=[
        jax.ShapeDtypeStruct((Q, CAP), jnp.float32),
        jax.ShapeDtypeStruct((Q, CAP), jnp.int32),
    ],
    scratch_types=[
        pltpu.VMEM((CHUNK,), jnp.float32),
        pltpu.VMEM((CAP + LANES,), jnp.float32),
        pltpu.VMEM((CAP + LANES,), jnp.int32),
        pltpu.VMEM((ROWS_PER_W + LANES,), jnp.float32),
    ],
)(_filter_body)


def kernel(queries, keys, doc_ids, W, b):
    q = _project(queries, W, b)
    sim, t4 = _sim(q, keys)
    u = _thresh(t4).reshape(Q)
    cvals, cidx = _filter(sim, u)
    tpos = _top100(cvals).T
    inds = jnp.take_along_axis(cidx, tpos, axis=1)
    chunks = jnp.take(keys, inds, axis=0)
    qn = q / (jnp.linalg.norm(q, axis=-1, keepdims=True) + 1e-8)
    cn = chunks / (jnp.linalg.norm(chunks, axis=-1, keepdims=True) + 1e-8)
    cos = jnp.sum(qn[:, None, :] * cn, axis=-1)
    scores = jnp.exp(cos)
    docs = jnp.take(doc_ids, inds, axis=0)
    row = jnp.arange(Q, dtype=jnp.int32)[:, None]
    doc_scores = jnp.zeros((Q, N_DOCS_PAD), dtype=jnp.float32)
    doc_scores = doc_scores.at[row, docs].add(scores)
    s_t, d_t = _top10(doc_scores)
    return s_t.T, d_t.T
